# baseline (speedup 1.0000x reference)
import jax
import jax.numpy as jnp
from jax import lax
from jax.experimental import pallas as pl
from jax.experimental.pallas import tpu as pltpu
from jax.experimental.pallas import tpu_sc as plsc

_N = 10000
_D = 128
_E = 320000
_EPS = 1e-5

_NC = 2
_NS = 16
_NW = _NC * _NS
_CH = 128
_KH = 16
_NPH = 160
_EPAD = _NPH * _KH * _CH
_PH0 = 5
_PH1 = 5
_ROW1 = _NS * _PH0
_PHD = 5
_RPT = 632
_NPAD = _NS * _RPT
_TRASH = 10008
_DEGW = 128
_NBUF = 2

_BLK = 400
_GRID = _N // _BLK


def _mesh():
    return plsc.VectorSubcoreMesh(
        core_axis_name="c", subcore_axis_name="s",
        num_cores=_NC, num_subcores=_NS)


def _deg_body(dst_hbm, ones_hbm, zdeg_hbm, out_hbm, dst_v, ones_v, acc, sem):
    c = lax.axis_index("c")
    s = lax.axis_index("s")
    w = c * _NS + s
    pltpu.sync_copy(zdeg_hbm, acc.at[pl.ds(s * _RPT, _RPT)])
    pltpu.sync_copy(ones_hbm, ones_v)
    plsc.subcore_barrier()

    for q in range(_PHD):
        pltpu.sync_copy(dst_hbm.at[w * _PHD + q], dst_v)

        @pl.loop(0, _KH)
        def _chunk(j):
            pltpu.sync_copy(ones_v, acc.at[dst_v.at[j]], add=True)

    plsc.subcore_barrier()
    pltpu.sync_copy(acc.at[pl.ds(s * _RPT, _RPT)],
                    out_hbm.at[pl.ds(c * _NPAD + s * _RPT, _RPT)])


def _deg_call(dst_p, ones, zdeg):
    fn = pl.kernel(
        _deg_body,
        out_type=jax.ShapeDtypeStruct((_NC * _NPAD, _DEGW), jnp.float32),
        mesh=_mesh(),
        scratch_types=[
            pltpu.VMEM((_KH, _CH), jnp.int32),
            pltpu.VMEM((_CH, _DEGW), jnp.float32),
            pltpu.VMEM_SHARED((_NPAD, _DEGW), jnp.float32),
            pltpu.SemaphoreType.DMA,
        ],
    )
    return fn(dst_p, ones, zdeg)


def _mp_body(ga_hbm, gb_hbm, src_hbm, dst_hbm, zrow_hbm, out0_hbm, out1_hbm,
             src_v, dst_v, r0, r1, acc, s0, s1):
    c = lax.axis_index("c")
    s = lax.axis_index("s")
    bufs = (r0, r1)
    sems = (s0, s1)
    pltpu.sync_copy(zrow_hbm, acc.at[pl.ds(s * _RPT, _RPT)])
    plsc.subcore_barrier()

    def _run_phase(g_hbm, row):
        pltpu.sync_copy(src_hbm.at[row], src_v)
        pltpu.sync_copy(dst_hbm.at[row], dst_v)

        for b in range(_NBUF):
            pltpu.async_copy(g_hbm.at[src_v.at[b]], bufs[b], sems[b])

        @pl.loop(0, _KH - _NBUF, step=_NBUF)
        def _grp(j):
            for b in range(_NBUF):
                pltpu.make_async_copy(g_hbm.at[src_v.at[j + b]],
                                      bufs[b], sems[b]).wait()
                pltpu.sync_copy(bufs[b], acc.at[dst_v.at[j + b]], add=True)
                pltpu.async_copy(g_hbm.at[src_v.at[j + b + _NBUF]],
                                 bufs[b], sems[b])

        for b in range(_NBUF):
            pltpu.make_async_copy(g_hbm.at[src_v.at[_KH - _NBUF + b]],
                                  bufs[b], sems[b]).wait()
            pltpu.sync_copy(bufs[b], acc.at[dst_v.at[_KH - _NBUF + b]],
                            add=True)

    @pl.when(c == 0)
    def _():
        for q in range(_PH0):
            _run_phase(ga_hbm, s * _PH0 + q)

    @pl.when(c == 1)
    def _():
        for q in range(_PH1):
            _run_phase(gb_hbm, _ROW1 + s * _PH1 + q)

    plsc.subcore_barrier()

    @pl.when(c == 0)
    def _():
        pltpu.sync_copy(acc.at[pl.ds(s * _RPT, _RPT)],
                        out0_hbm.at[pl.ds(s * _RPT, _RPT)])

    @pl.when(c == 1)
    def _():
        pltpu.sync_copy(acc.at[pl.ds(s * _RPT, _RPT)],
                        out1_hbm.at[pl.ds(s * _RPT, _RPT)])


def _mp_call(ga, gb, src_p, dst_p, zrow):
    fn = pl.kernel(
        _mp_body,
        out_type=[jax.ShapeDtypeStruct((_NPAD, _D), jnp.float32),
                  jax.ShapeDtypeStruct((_NPAD, _D), jnp.float32)],
        mesh=_mesh(),
        scratch_types=[
            pltpu.VMEM((_KH, _CH), jnp.int32),
            pltpu.VMEM((_KH, _CH), jnp.int32),
            pltpu.VMEM((_CH, _D), jnp.float32),
            pltpu.VMEM((_CH, _D), jnp.float32),
            pltpu.VMEM_SHARED((_NPAD, _D), jnp.float32),
            pltpu.SemaphoreType.DMA,
            pltpu.SemaphoreType.DMA,
        ],
    )
    return fn(ga, gb, src_p, dst_p, zrow)


def _prelude_body(x_ref, w_ref, d0_ref, d1_ref, h_ref, g_ref, dinv_ref, sw_ref):
    deg = d0_ref[...] + d1_ref[...] + 1.0
    dinv = lax.rsqrt(deg)
    h = jnp.dot(x_ref[...], w_ref[...], preferred_element_type=jnp.float32)
    h_ref[...] = h
    g_ref[...] = h * dinv
    dinv_ref[...] = dinv
    sw_ref[...] = 1.0 / deg


def _prelude_call(x, W0, d0, d1):
    row = pl.BlockSpec((_BLK, _D), lambda i: (i, 0))
    col = pl.BlockSpec((_BLK, 1), lambda i: (i, 0))
    return pl.pallas_call(
        _prelude_body,
        grid=(_GRID,),
        in_specs=[row, pl.BlockSpec((_D, _D), lambda i: (0, 0)), col, col],
        out_specs=[row, row, col, col],
        out_shape=[
            jax.ShapeDtypeStruct((_N, _D), jnp.float32),
            jax.ShapeDtypeStruct((_N, _D), jnp.float32),
            jax.ShapeDtypeStruct((_N, 1), jnp.float32),
            jax.ShapeDtypeStruct((_N, 1), jnp.float32),
        ],
    )(x, W0, d0, d1)


def _stats_body(p0_ref, p1_ref, h_ref, dinv_ref, sw_ref, b_ref,
                out_ref, s1_ref, s2_ref):
    o = (dinv_ref[...] * (p0_ref[...] + p1_ref[...])
         + h_ref[...] * sw_ref[...] + b_ref[...])
    out_ref[...] = o

    @pl.when(pl.program_id(0) == 0)
    def _():
        s1_ref[...] = jnp.zeros_like(s1_ref)
        s2_ref[...] = jnp.zeros_like(s2_ref)

    s1_ref[...] += jnp.sum(o, axis=0, keepdims=True)
    s2_ref[...] += jnp.sum(o * o, axis=0, keepdims=True)


def _stats_call(p0, p1, h, dinv, sw, b):
    row = pl.BlockSpec((_BLK, _D), lambda i: (i, 0))
    col = pl.BlockSpec((_BLK, 1), lambda i: (i, 0))
    vec = pl.BlockSpec((1, _D), lambda i: (0, 0))
    return pl.pallas_call(
        _stats_body,
        grid=(_GRID,),
        in_specs=[row, row, row, col, col, vec],
        out_specs=[row, vec, vec],
        out_shape=[
            jax.ShapeDtypeStruct((_N, _D), jnp.float32),
            jax.ShapeDtypeStruct((1, _D), jnp.float32),
            jax.ShapeDtypeStruct((1, _D), jnp.float32),
        ],
    )(p0, p1, h, dinv, sw, b)


def _apply_body(o_ref, s1_ref, s2_ref, gam_ref, bet_ref, w_ref, dinv_ref,
                h_ref, g_ref):
    mean = s1_ref[...] / _N
    var = s2_ref[...] / _N - mean * mean
    istd = lax.rsqrt(var + _EPS)
    z = (o_ref[...] - mean) * istd * gam_ref[...] + bet_ref[...]
    z = jnp.maximum(z, 0.0)
    h = jnp.dot(z, w_ref[...], preferred_element_type=jnp.float32)
    h_ref[...] = h
    g_ref[...] = h * dinv_ref[...]


def _apply_call(o, s1, s2, gamma, beta, W, dinv):
    row = pl.BlockSpec((_BLK, _D), lambda i: (i, 0))
    col = pl.BlockSpec((_BLK, 1), lambda i: (i, 0))
    vec = pl.BlockSpec((1, _D), lambda i: (0, 0))
    return pl.pallas_call(
        _apply_body,
        grid=(_GRID,),
        in_specs=[row, vec, vec, vec, vec,
                  pl.BlockSpec((_D, _D), lambda i: (0, 0)), col],
        out_specs=[row, row],
        out_shape=[
            jax.ShapeDtypeStruct((_N, _D), jnp.float32),
            jax.ShapeDtypeStruct((_N, _D), jnp.float32),
        ],
    )(o, s1, s2, gamma, beta, W, dinv)


def _final_body(p0_ref, p1_ref, h_ref, dinv_ref, sw_ref, b_ref, out_ref):
    out_ref[...] = (dinv_ref[...] * (p0_ref[...] + p1_ref[...])
                    + h_ref[...] * sw_ref[...] + b_ref[...])


def _final_call(p0, p1, h, dinv, sw, b):
    row = pl.BlockSpec((_BLK, _D), lambda i: (i, 0))
    col = pl.BlockSpec((_BLK, 1), lambda i: (i, 0))
    vec = pl.BlockSpec((1, _D), lambda i: (0, 0))
    return pl.pallas_call(
        _final_body,
        grid=(_GRID,),
        in_specs=[row, row, row, col, col, vec],
        out_specs=row,
        out_shape=jax.ShapeDtypeStruct((_N, _D), jnp.float32),
    )(p0, p1, h, dinv, sw, b)


def kernel(x, edge_index, W0, b0, W1, b1, W2, b2, W3, b3,
           gamma0, beta0, gamma1, beta1, gamma2, beta2):
    src = edge_index[0]
    dst = edge_index[1]
    pad = _EPAD - _E
    fill = jnp.arange(pad, dtype=jnp.int32)
    src_p = jnp.concatenate(
        [src, fill % _N]).reshape(_NPH, _KH, _CH)
    dst_p = jnp.concatenate(
        [dst, _N + fill % (_NPAD - _N)]).reshape(_NPH, _KH, _CH)
    ones = jnp.ones((_CH, _DEGW), jnp.float32)
    zdeg = jnp.zeros((_RPT, _DEGW), jnp.float32)
    zrow = jnp.zeros((_RPT, _D), jnp.float32)

    deg_out = _deg_call(dst_p, ones, zdeg)
    d0 = deg_out[:_N, 0:1]
    d1 = deg_out[_NPAD:_NPAD + _N, 0:1]

    h, g, dinv, selfw = _prelude_call(x, W0, d0, d1)

    Ws = [W1, W2, W3]
    bs = [b0, b1, b2]
    gammas = [gamma0, gamma1, gamma2]
    betas = [beta0, beta1, beta2]
    for i in range(3):
        p0, p1 = _mp_call(g, g, src_p, dst_p, zrow)
        out, s1, s2 = _stats_call(p0, p1, h, dinv, selfw,
                                  bs[i].reshape(1, _D))
        h, g = _apply_call(out, s1, s2, gammas[i].reshape(1, _D),
                           betas[i].reshape(1, _D), Ws[i], dinv)
    p0, p1 = _mp_call(g, g, src_p, dst_p, zrow)
    return _final_call(p0, p1, h, dinv, selfw,
                       b3.reshape(1, _D))


# speedup vs baseline: 1.0652x; 1.0652x over previous
import jax
import jax.numpy as jnp
from jax import lax
from jax.experimental import pallas as pl
from jax.experimental.pallas import tpu as pltpu
from jax.experimental.pallas import tpu_sc as plsc

_N = 10000
_D = 128
_E = 320000
_EPS = 1e-5

_NC = 2
_NS = 16
_NW = _NC * _NS
_CH = 64
_KH = 32
_NPH = 160
_EPAD = _NPH * _KH * _CH
_PH0 = 5
_PH1 = 5
_ROW1 = _NS * _PH0
_PHD = 5
_RPT = 632
_NPAD = _NS * _RPT
_TRASH = 10008
_DEGW = 128
_NBUF = 4

_BLK = 400
_GRID = _N // _BLK


def _mesh():
    return plsc.VectorSubcoreMesh(
        core_axis_name="c", subcore_axis_name="s",
        num_cores=_NC, num_subcores=_NS)


def _deg_body(dst_hbm, ones_hbm, zdeg_hbm, out_hbm, dst_v, ones_v, acc, sem):
    c = lax.axis_index("c")
    s = lax.axis_index("s")
    w = c * _NS + s
    pltpu.sync_copy(zdeg_hbm, acc.at[pl.ds(s * _RPT, _RPT)])
    pltpu.sync_copy(ones_hbm, ones_v)
    plsc.subcore_barrier()

    for q in range(_PHD):
        pltpu.sync_copy(dst_hbm.at[w * _PHD + q], dst_v)

        @pl.loop(0, _KH)
        def _chunk(j):
            pltpu.sync_copy(ones_v, acc.at[dst_v.at[j]], add=True)

    plsc.subcore_barrier()
    pltpu.sync_copy(acc.at[pl.ds(s * _RPT, _RPT)],
                    out_hbm.at[pl.ds(c * _NPAD + s * _RPT, _RPT)])


def _deg_call(dst_p, ones, zdeg):
    fn = pl.kernel(
        _deg_body,
        out_type=jax.ShapeDtypeStruct((_NC * _NPAD, _DEGW), jnp.float32),
        mesh=_mesh(),
        scratch_types=[
            pltpu.VMEM((_KH, _CH), jnp.int32),
            pltpu.VMEM((_CH, _DEGW), jnp.float32),
            pltpu.VMEM_SHARED((_NPAD, _DEGW), jnp.float32),
            pltpu.SemaphoreType.DMA,
        ],
    )
    return fn(dst_p, ones, zdeg)


def _mp_body(ga_hbm, gb_hbm, src_hbm, dst_hbm, zrow_hbm, out0_hbm, out1_hbm,
             src_v, dst_v, r0, r1, r2, r3, acc, s0, s1, s2, s3):
    c = lax.axis_index("c")
    s = lax.axis_index("s")
    bufs = (r0, r1, r2, r3)
    sems = (s0, s1, s2, s3)
    pltpu.sync_copy(zrow_hbm, acc.at[pl.ds(s * _RPT, _RPT)])
    plsc.subcore_barrier()

    def _run_phase(g_hbm, row):
        pltpu.sync_copy(src_hbm.at[row], src_v)
        pltpu.sync_copy(dst_hbm.at[row], dst_v)

        for b in range(_NBUF):
            pltpu.async_copy(g_hbm.at[src_v.at[b]], bufs[b], sems[b])

        @pl.loop(0, _KH - _NBUF, step=_NBUF)
        def _grp(j):
            for b in range(_NBUF):
                pltpu.make_async_copy(g_hbm.at[src_v.at[j + b]],
                                      bufs[b], sems[b]).wait()
                pltpu.sync_copy(bufs[b], acc.at[dst_v.at[j + b]], add=True)
                pltpu.async_copy(g_hbm.at[src_v.at[j + b + _NBUF]],
                                 bufs[b], sems[b])

        for b in range(_NBUF):
            pltpu.make_async_copy(g_hbm.at[src_v.at[_KH - _NBUF + b]],
                                  bufs[b], sems[b]).wait()
            pltpu.sync_copy(bufs[b], acc.at[dst_v.at[_KH - _NBUF + b]],
                            add=True)

    @pl.when(c == 0)
    def _():
        for q in range(_PH0):
            _run_phase(ga_hbm, s * _PH0 + q)

    @pl.when(c == 1)
    def _():
        for q in range(_PH1):
            _run_phase(gb_hbm, _ROW1 + s * _PH1 + q)

    plsc.subcore_barrier()

    @pl.when(c == 0)
    def _():
        pltpu.sync_copy(acc.at[pl.ds(s * _RPT, _RPT)],
                        out0_hbm.at[pl.ds(s * _RPT, _RPT)])

    @pl.when(c == 1)
    def _():
        pltpu.sync_copy(acc.at[pl.ds(s * _RPT, _RPT)],
                        out1_hbm.at[pl.ds(s * _RPT, _RPT)])


def _mp_call(ga, gb, src_p, dst_p, zrow):
    fn = pl.kernel(
        _mp_body,
        out_type=[jax.ShapeDtypeStruct((_NPAD, _D), jnp.float32),
                  jax.ShapeDtypeStruct((_NPAD, _D), jnp.float32)],
        mesh=_mesh(),
        scratch_types=[
            pltpu.VMEM((_KH, _CH), jnp.int32),
            pltpu.VMEM((_KH, _CH), jnp.int32),
            pltpu.VMEM((_CH, _D), jnp.float32),
            pltpu.VMEM((_CH, _D), jnp.float32),
            pltpu.VMEM((_CH, _D), jnp.float32),
            pltpu.VMEM((_CH, _D), jnp.float32),
            pltpu.VMEM_SHARED((_NPAD, _D), jnp.float32),
            pltpu.SemaphoreType.DMA,
            pltpu.SemaphoreType.DMA,
            pltpu.SemaphoreType.DMA,
            pltpu.SemaphoreType.DMA,
        ],
    )
    return fn(ga, gb, src_p, dst_p, zrow)


def _prelude_body(x_ref, w_ref, d0_ref, d1_ref, h_ref, g_ref, dinv_ref, sw_ref):
    deg = d0_ref[...] + d1_ref[...] + 1.0
    dinv = lax.rsqrt(deg)
    h = jnp.dot(x_ref[...], w_ref[...], preferred_element_type=jnp.float32)
    h_ref[...] = h
    g_ref[...] = h * dinv
    dinv_ref[...] = dinv
    sw_ref[...] = 1.0 / deg


def _prelude_call(x, W0, d0, d1):
    row = pl.BlockSpec((_BLK, _D), lambda i: (i, 0))
    col = pl.BlockSpec((_BLK, 1), lambda i: (i, 0))
    return pl.pallas_call(
        _prelude_body,
        grid=(_GRID,),
        in_specs=[row, pl.BlockSpec((_D, _D), lambda i: (0, 0)), col, col],
        out_specs=[row, row, col, col],
        out_shape=[
            jax.ShapeDtypeStruct((_N, _D), jnp.float32),
            jax.ShapeDtypeStruct((_N, _D), jnp.float32),
            jax.ShapeDtypeStruct((_N, 1), jnp.float32),
            jax.ShapeDtypeStruct((_N, 1), jnp.float32),
        ],
    )(x, W0, d0, d1)


def _layer_body(p0_ref, p1_ref, h_ref, dinv_ref, sw_ref, b_ref,
                gam_ref, bet_ref, w_ref, h_out, g_out, out_s, s1_s, s2_s):
    t = pl.program_id(0)
    i = pl.program_id(1)

    @pl.when(t == 0)
    def _():
        o = (dinv_ref[...] * (p0_ref[...] + p1_ref[...])
             + h_ref[...] * sw_ref[...] + b_ref[...])
        out_s[pl.ds(i * _BLK, _BLK), :] = o

        @pl.when(i == 0)
        def _():
            s1_s[...] = jnp.zeros_like(s1_s)
            s2_s[...] = jnp.zeros_like(s2_s)

        s1_s[...] += jnp.sum(o, axis=0, keepdims=True)
        s2_s[...] += jnp.sum(o * o, axis=0, keepdims=True)

    @pl.when(t == 1)
    def _():
        mean = s1_s[...] / _N
        var = s2_s[...] / _N - mean * mean
        istd = lax.rsqrt(var + _EPS)
        z = (out_s[pl.ds(i * _BLK, _BLK), :] - mean) * istd * gam_ref[...] \
            + bet_ref[...]
        z = jnp.maximum(z, 0.0)
        hn = jnp.dot(z, w_ref[...], preferred_element_type=jnp.float32)
        h_out[...] = hn
        g_out[...] = hn * dinv_ref[...]


def _layer_call(p0, p1, h, dinv, sw, b, gamma, beta, W):
    rowp0 = pl.BlockSpec((_BLK, _D), lambda t, i: (jnp.where(t == 0, i, 0), 0))
    rowp1 = pl.BlockSpec((_BLK, _D), lambda t, i: (jnp.where(t == 1, i, 0), 0))
    col = pl.BlockSpec((_BLK, 1), lambda t, i: (i, 0))
    vec = pl.BlockSpec((1, _D), lambda t, i: (0, 0))
    full = pl.BlockSpec((_D, _D), lambda t, i: (0, 0))
    return pl.pallas_call(
        _layer_body,
        grid=(2, _GRID),
        in_specs=[rowp0, rowp0, rowp0, col, col, vec, vec, vec, full],
        out_specs=[rowp1, rowp1],
        out_shape=[
            jax.ShapeDtypeStruct((_N, _D), jnp.float32),
            jax.ShapeDtypeStruct((_N, _D), jnp.float32),
        ],
        scratch_shapes=[
            pltpu.VMEM((_N, _D), jnp.float32),
            pltpu.VMEM((1, _D), jnp.float32),
            pltpu.VMEM((1, _D), jnp.float32),
        ],
    )(p0, p1, h, dinv, sw, b, gamma, beta, W)


def _final_body(p0_ref, p1_ref, h_ref, dinv_ref, sw_ref, b_ref, out_ref):
    out_ref[...] = (dinv_ref[...] * (p0_ref[...] + p1_ref[...])
                    + h_ref[...] * sw_ref[...] + b_ref[...])


def _final_call(p0, p1, h, dinv, sw, b):
    row = pl.BlockSpec((_BLK, _D), lambda i: (i, 0))
    col = pl.BlockSpec((_BLK, 1), lambda i: (i, 0))
    vec = pl.BlockSpec((1, _D), lambda i: (0, 0))
    return pl.pallas_call(
        _final_body,
        grid=(_GRID,),
        in_specs=[row, row, row, col, col, vec],
        out_specs=row,
        out_shape=jax.ShapeDtypeStruct((_N, _D), jnp.float32),
    )(p0, p1, h, dinv, sw, b)


def kernel(x, edge_index, W0, b0, W1, b1, W2, b2, W3, b3,
           gamma0, beta0, gamma1, beta1, gamma2, beta2):
    src = edge_index[0]
    dst = edge_index[1]
    pad = _EPAD - _E
    fill = jnp.arange(pad, dtype=jnp.int32)
    src_p = jnp.concatenate(
        [src, fill % _N]).reshape(_NPH, _KH, _CH)
    dst_p = jnp.concatenate(
        [dst, _N + fill % (_NPAD - _N)]).reshape(_NPH, _KH, _CH)
    ones = jnp.ones((_CH, _DEGW), jnp.float32)
    zdeg = jnp.zeros((_RPT, _DEGW), jnp.float32)
    zrow = jnp.zeros((_RPT, _D), jnp.float32)

    deg_out = _deg_call(dst_p, ones, zdeg)
    d0 = deg_out[:_N, 0:1]
    d1 = deg_out[_NPAD:_NPAD + _N, 0:1]

    h, g, dinv, selfw = _prelude_call(x, W0, d0, d1)

    Ws = [W1, W2, W3]
    bs = [b0, b1, b2]
    gammas = [gamma0, gamma1, gamma2]
    betas = [beta0, beta1, beta2]
    for i in range(3):
        p0, p1 = _mp_call(g, g, src_p, dst_p, zrow)
        h, g = _layer_call(p0, p1, h, dinv, selfw, bs[i].reshape(1, _D),
                           gammas[i].reshape(1, _D), betas[i].reshape(1, _D),
                           Ws[i])
    p0, p1 = _mp_call(g, g, src_p, dst_p, zrow)
    return _final_call(p0, p1, h, dinv, selfw,
                       b3.reshape(1, _D))


# speedup vs baseline: 1.1966x; 1.1233x over previous
import jax
import jax.numpy as jnp
from jax import lax
from jax.experimental import pallas as pl
from jax.experimental.pallas import tpu as pltpu
from jax.experimental.pallas import tpu_sc as plsc

_N = 10000
_D = 128
_E = 320000
_EPS = 1e-5

_NC = 2
_NS = 16
_NW = _NC * _NS
_CH = 64
_KH = 32
_NPH = 160
_EPAD = _NPH * _KH * _CH
_PH0 = 5
_PH1 = 5
_ROW1 = _NS * _PH0
_PHD = 5
_RPT = 632
_NPAD = _NS * _RPT
_TRASH = 10008
_DEGW = 128
_NBUF = 4

_BLK = 2000
_GRID = _N // _BLK


def _mesh():
    return plsc.VectorSubcoreMesh(
        core_axis_name="c", subcore_axis_name="s",
        num_cores=_NC, num_subcores=_NS)


def _deg_body(dst_hbm, ones_hbm, zdeg_hbm, out_hbm, dst_v, ones_v, acc, sem):
    c = lax.axis_index("c")
    s = lax.axis_index("s")
    w = c * _NS + s
    pltpu.sync_copy(zdeg_hbm, acc.at[pl.ds(s * _RPT, _RPT)])
    pltpu.sync_copy(ones_hbm, ones_v)
    plsc.subcore_barrier()

    for q in range(_PHD):
        pltpu.sync_copy(dst_hbm.at[w * _PHD + q], dst_v)

        @pl.loop(0, _KH)
        def _chunk(j):
            pltpu.sync_copy(ones_v, acc.at[dst_v.at[j]], add=True)

    plsc.subcore_barrier()
    pltpu.sync_copy(acc.at[pl.ds(s * _RPT, _RPT)],
                    out_hbm.at[pl.ds(c * _NPAD + s * _RPT, _RPT)])


def _deg_call(dst_p, ones, zdeg):
    fn = pl.kernel(
        _deg_body,
        out_type=jax.ShapeDtypeStruct((_NC * _NPAD, _DEGW), jnp.float32),
        mesh=_mesh(),
        scratch_types=[
            pltpu.VMEM((_KH, _CH), jnp.int32),
            pltpu.VMEM((_CH, _DEGW), jnp.float32),
            pltpu.VMEM_SHARED((_NPAD, _DEGW), jnp.float32),
            pltpu.SemaphoreType.DMA,
        ],
    )
    return fn(dst_p, ones, zdeg)


def _mp_body(ga_hbm, gb_hbm, src_hbm, dst_hbm, zrow_hbm, out0_hbm, out1_hbm,
             src_v, dst_v, r0, r1, r2, r3, acc, s0, s1, s2, s3):
    c = lax.axis_index("c")
    s = lax.axis_index("s")
    bufs = (r0, r1, r2, r3)
    sems = (s0, s1, s2, s3)
    pltpu.sync_copy(zrow_hbm, acc.at[pl.ds(s * _RPT, _RPT)])
    plsc.subcore_barrier()

    def _run_phase(g_hbm, row):
        pltpu.sync_copy(src_hbm.at[row], src_v)
        pltpu.sync_copy(dst_hbm.at[row], dst_v)

        for b in range(_NBUF):
            pltpu.async_copy(g_hbm.at[src_v.at[b]], bufs[b], sems[b])

        @pl.loop(0, _KH - _NBUF, step=_NBUF)
        def _grp(j):
            for b in range(_NBUF):
                pltpu.make_async_copy(g_hbm.at[src_v.at[j + b]],
                                      bufs[b], sems[b]).wait()
                pltpu.sync_copy(bufs[b], acc.at[dst_v.at[j + b]], add=True)
                pltpu.async_copy(g_hbm.at[src_v.at[j + b + _NBUF]],
                                 bufs[b], sems[b])

        for b in range(_NBUF):
            pltpu.make_async_copy(g_hbm.at[src_v.at[_KH - _NBUF + b]],
                                  bufs[b], sems[b]).wait()
            pltpu.sync_copy(bufs[b], acc.at[dst_v.at[_KH - _NBUF + b]],
                            add=True)

    @pl.when(c == 0)
    def _():
        for q in range(_PH0):
            _run_phase(ga_hbm, s * _PH0 + q)

    @pl.when(c == 1)
    def _():
        for q in range(_PH1):
            _run_phase(gb_hbm, _ROW1 + s * _PH1 + q)

    plsc.subcore_barrier()

    @pl.when(c == 0)
    def _():
        pltpu.sync_copy(acc.at[pl.ds(s * _RPT, _RPT)],
                        out0_hbm.at[pl.ds(s * _RPT, _RPT)])

    @pl.when(c == 1)
    def _():
        pltpu.sync_copy(acc.at[pl.ds(s * _RPT, _RPT)],
                        out1_hbm.at[pl.ds(s * _RPT, _RPT)])


def _mp_call(ga, gb, src_p, dst_p, zrow):
    fn = pl.kernel(
        _mp_body,
        out_type=[jax.ShapeDtypeStruct((_NPAD, _D), jnp.float32),
                  jax.ShapeDtypeStruct((_NPAD, _D), jnp.float32)],
        mesh=_mesh(),
        scratch_types=[
            pltpu.VMEM((_KH, _CH), jnp.int32),
            pltpu.VMEM((_KH, _CH), jnp.int32),
            pltpu.VMEM((_CH, _D), jnp.float32),
            pltpu.VMEM((_CH, _D), jnp.float32),
            pltpu.VMEM((_CH, _D), jnp.float32),
            pltpu.VMEM((_CH, _D), jnp.float32),
            pltpu.VMEM_SHARED((_NPAD, _D), jnp.float32),
            pltpu.SemaphoreType.DMA,
            pltpu.SemaphoreType.DMA,
            pltpu.SemaphoreType.DMA,
            pltpu.SemaphoreType.DMA,
        ],
    )
    return fn(ga, gb, src_p, dst_p, zrow)


def _prelude_body(x_ref, w_ref, d0_ref, d1_ref, h_ref, g_ref, dinv_ref, sw_ref):
    deg = d0_ref[...] + d1_ref[...] + 1.0
    dinv = lax.rsqrt(deg)
    h = jnp.dot(x_ref[...], w_ref[...], preferred_element_type=jnp.float32)
    h_ref[...] = h
    g_ref[...] = h * dinv
    dinv_ref[...] = dinv
    sw_ref[...] = 1.0 / deg


def _prelude_call(x, W0, d0, d1):
    row = pl.BlockSpec((_BLK, _D), lambda i: (i, 0))
    col = pl.BlockSpec((_BLK, 1), lambda i: (i, 0))
    return pl.pallas_call(
        _prelude_body,
        grid=(_GRID,),
        in_specs=[row, pl.BlockSpec((_D, _D), lambda i: (0, 0)), col, col],
        out_specs=[row, row, col, col],
        out_shape=[
            jax.ShapeDtypeStruct((_N, _D), jnp.float32),
            jax.ShapeDtypeStruct((_N, _D), jnp.float32),
            jax.ShapeDtypeStruct((_N, 1), jnp.float32),
            jax.ShapeDtypeStruct((_N, 1), jnp.float32),
        ],
    )(x, W0, d0, d1)


def _layer_body(p0_ref, p1_ref, h_ref, dinv_ref, sw_ref, b_ref,
                gam_ref, bet_ref, w_ref, h_out, g_out, out_s, s1_s, s2_s):
    t = pl.program_id(0)
    i = pl.program_id(1)

    @pl.when(t == 0)
    def _():
        o = (dinv_ref[...] * (p0_ref[...] + p1_ref[...])
             + h_ref[...] * sw_ref[...] + b_ref[...])
        out_s[pl.ds(i * _BLK, _BLK), :] = o

        @pl.when(i == 0)
        def _():
            s1_s[...] = jnp.zeros_like(s1_s)
            s2_s[...] = jnp.zeros_like(s2_s)

        s1_s[...] += jnp.sum(o, axis=0, keepdims=True)
        s2_s[...] += jnp.sum(o * o, axis=0, keepdims=True)

    @pl.when(t == 1)
    def _():
        mean = s1_s[...] / _N
        var = s2_s[...] / _N - mean * mean
        istd = lax.rsqrt(var + _EPS)
        z = (out_s[pl.ds(i * _BLK, _BLK), :] - mean) * istd * gam_ref[...] \
            + bet_ref[...]
        z = jnp.maximum(z, 0.0)
        hn = jnp.dot(z, w_ref[...], preferred_element_type=jnp.float32)
        h_out[...] = hn
        g_out[...] = hn * dinv_ref[...]


def _layer_call(p0, p1, h, dinv, sw, b, gamma, beta, W):
    rowp0 = pl.BlockSpec((_BLK, _D), lambda t, i: (jnp.where(t == 0, i, 0), 0))
    rowp1 = pl.BlockSpec((_BLK, _D), lambda t, i: (jnp.where(t == 1, i, 0), 0))
    col = pl.BlockSpec((_BLK, 1), lambda t, i: (i, 0))
    vec = pl.BlockSpec((1, _D), lambda t, i: (0, 0))
    full = pl.BlockSpec((_D, _D), lambda t, i: (0, 0))
    return pl.pallas_call(
        _layer_body,
        grid=(2, _GRID),
        in_specs=[rowp0, rowp0, rowp0, col, col, vec, vec, vec, full],
        out_specs=[rowp1, rowp1],
        out_shape=[
            jax.ShapeDtypeStruct((_N, _D), jnp.float32),
            jax.ShapeDtypeStruct((_N, _D), jnp.float32),
        ],
        scratch_shapes=[
            pltpu.VMEM((_N, _D), jnp.float32),
            pltpu.VMEM((1, _D), jnp.float32),
            pltpu.VMEM((1, _D), jnp.float32),
        ],
    )(p0, p1, h, dinv, sw, b, gamma, beta, W)


def _final_body(p0_ref, p1_ref, h_ref, dinv_ref, sw_ref, b_ref, out_ref):
    out_ref[...] = (dinv_ref[...] * (p0_ref[...] + p1_ref[...])
                    + h_ref[...] * sw_ref[...] + b_ref[...])


def _final_call(p0, p1, h, dinv, sw, b):
    row = pl.BlockSpec((_BLK, _D), lambda i: (i, 0))
    col = pl.BlockSpec((_BLK, 1), lambda i: (i, 0))
    vec = pl.BlockSpec((1, _D), lambda i: (0, 0))
    return pl.pallas_call(
        _final_body,
        grid=(_GRID,),
        in_specs=[row, row, row, col, col, vec],
        out_specs=row,
        out_shape=jax.ShapeDtypeStruct((_N, _D), jnp.float32),
    )(p0, p1, h, dinv, sw, b)


def kernel(x, edge_index, W0, b0, W1, b1, W2, b2, W3, b3,
           gamma0, beta0, gamma1, beta1, gamma2, beta2):
    src = edge_index[0]
    dst = edge_index[1]
    pad = _EPAD - _E
    fill = jnp.arange(pad, dtype=jnp.int32)
    src_p = jnp.concatenate(
        [src, fill % _N]).reshape(_NPH, _KH, _CH)
    dst_p = jnp.concatenate(
        [dst, _N + fill % (_NPAD - _N)]).reshape(_NPH, _KH, _CH)
    ones = jnp.ones((_CH, _DEGW), jnp.float32)
    zdeg = jnp.zeros((_RPT, _DEGW), jnp.float32)
    zrow = jnp.zeros((_RPT, _D), jnp.float32)

    deg_out = _deg_call(dst_p, ones, zdeg)
    d0 = deg_out[:_N, 0:1]
    d1 = deg_out[_NPAD:_NPAD + _N, 0:1]

    h, g, dinv, selfw = _prelude_call(x, W0, d0, d1)

    Ws = [W1, W2, W3]
    bs = [b0, b1, b2]
    gammas = [gamma0, gamma1, gamma2]
    betas = [beta0, beta1, beta2]
    for i in range(3):
        p0, p1 = _mp_call(g, g, src_p, dst_p, zrow)
        h, g = _layer_call(p0, p1, h, dinv, selfw, bs[i].reshape(1, _D),
                           gammas[i].reshape(1, _D), betas[i].reshape(1, _D),
                           Ws[i])
    p0, p1 = _mp_call(g, g, src_p, dst_p, zrow)
    return _final_call(p0, p1, h, dinv, selfw,
                       b3.reshape(1, _D))
